# baseline (speedup 1.0000x reference)
import numpy as np
import jax
import jax.numpy as jnp
from jax.experimental import pallas as pl
from jax.experimental.pallas import tpu as pltpu

_B = 32
_V = 1000000
_BLK = 32768
_NSTEPS = (_V + _BLK - 1) // _BLK
_CH = 256
_NCH = _BLK // _CH
_FULL = _V // _BLK
_TAILC = -(-(_V - _FULL * _BLK) // _CH)

_KS0 = 0
_KS1 = 42
_KS2 = (0x1BD11BDA ^ 42) & 0xFFFFFFFF
_ROTS = (13, 15, 26, 6, 17, 29, 16, 24, 13, 15, 26, 6, 17, 29, 16, 24,
         13, 15, 26, 6)
_INJ = {
    4: (_KS1, (_KS2 + 1) & 0xFFFFFFFF),
    8: (_KS2, (_KS0 + 2) & 0xFFFFFFFF),
    12: (_KS0, (_KS1 + 3) & 0xFFFFFFFF),
    16: (_KS1, (_KS2 + 4) & 0xFFFFFFFF),
    20: (_KS2, (_KS0 + 5) & 0xFFFFFFFF),
}

_TINY = np.float32(np.finfo(np.float32).tiny)


def _threefry_xor(c):
    x1 = c + jnp.uint32(_KS1)
    x0 = x1
    r = _ROTS[0]
    x1 = ((x1 << jnp.uint32(r)) | (x1 >> jnp.uint32(32 - r))) ^ x0
    for rnum, r in enumerate(_ROTS[1:], start=2):
        x0 = x0 + x1
        x1 = (x1 << jnp.uint32(r)) | (x1 >> jnp.uint32(32 - r))
        x1 = x1 ^ x0
        if rnum % 4 == 0:
            inj0, inj1 = _INJ[rnum]
            if inj0:
                x0 = x0 + jnp.uint32(inj0)
            x1 = x1 + jnp.uint32(inj1)
    return x0 ^ x1


def _neg_gumbel_from_bits(bits):
    fb = (bits >> jnp.uint32(9)) | jnp.uint32(0x3F800000)
    f = jax.lax.bitcast_convert_type(fb, jnp.float32) - jnp.float32(1.0)
    u = f + _TINY
    return jnp.log(-jnp.log(u))


def _body(a_ref, x_ref, samp_ref, lp_ref, mode_ref,
          rv_ref, s_ref, vm_ref, vi_ref, gm_ref, gi_ref, av_ref):
    step = pl.program_id(0)
    neg_inf = jnp.float32(-jnp.inf)

    @pl.when(step == 0)
    def _init():
        rv_ref[...] = (jax.lax.broadcasted_iota(jnp.int32, (_B, _CH), 0) * _V
                       + jax.lax.broadcasted_iota(jnp.int32, (_B, _CH), 1))
        s_ref[...] = jnp.zeros_like(s_ref)
        vm_ref[...] = jnp.full_like(vm_ref, neg_inf)
        vi_ref[...] = jnp.zeros_like(vi_ref)
        gm_ref[...] = jnp.full_like(gm_ref, neg_inf)
        gi_ref[...] = jnp.zeros_like(gi_ref)
        av_ref[...] = jnp.zeros_like(av_ref)

    base = step * _BLK
    rvl = rv_ref[...]
    ta = a_ref[...] + rv_ref[:, 0:1]

    def sweep(nch, masked):
        s = s_ref[...]
        vm = vm_ref[...]
        vi = vi_ref[...]
        gm = gm_ref[...]
        gi = gi_ref[...]
        av = av_ref[...]
        for c in range(nch):
            x = x_ref[:, c * _CH:(c + 1) * _CH]
            cnt = rvl + (base + c * _CH)
            if masked:
                x = jnp.where(
                    jax.lax.broadcasted_iota(jnp.int32, (_B, _CH), 1)
                    + (base + c * _CH) < _V, x, neg_inf)
            ng = _neg_gumbel_from_bits(_threefry_xor(cnt.astype(jnp.uint32)))
            y = x - ng
            gu = y > gm
            gm = jnp.where(gu, y, gm)
            gi = jnp.where(gu, cnt, gi)
            vu = x > vm
            vm = jnp.where(vu, x, vm)
            vi = jnp.where(vu, cnt, vi)
            s = s + jnp.exp(x)
            av = jnp.where(cnt == ta, x, av)
        s_ref[...] = s
        vm_ref[...] = vm
        vi_ref[...] = vi
        gm_ref[...] = gm
        gi_ref[...] = gi
        av_ref[...] = av

    @pl.when(step < _FULL)
    def _fast():
        sweep(_NCH, masked=False)

    @pl.when(step == _FULL)
    def _tail():
        sweep(_TAILC, masked=True)

    @pl.when(step == _NSTEPS - 1)
    def _fin():
        rv0 = rv_ref[:, 0:1]
        vm = vm_ref[...]
        bv = jnp.max(vm, axis=1, keepdims=True)
        mode_ref[...] = jnp.min(
            jnp.where(vm == bv, vi_ref[...], jnp.int32(0x7FFFFFFF)),
            axis=1, keepdims=True) - rv0
        gm = gm_ref[...]
        gv = jnp.max(gm, axis=1, keepdims=True)
        samp_ref[...] = jnp.min(
            jnp.where(gm == gv, gi_ref[...], jnp.int32(0x7FFFFFFF)),
            axis=1, keepdims=True) - rv0
        s = jnp.sum(s_ref[...], axis=1, keepdims=True)
        lp_ref[...] = (jnp.sum(av_ref[...], axis=1, keepdims=True)
                       - jnp.log(s))


def kernel(logits, actions):
    sample, log_probs, mode = pl.pallas_call(
        _body,
        grid=(_NSTEPS,),
        in_specs=[
            pl.BlockSpec((_B, 1), lambda i: (0, 0)),
            pl.BlockSpec((_B, _BLK), lambda i: (0, i)),
        ],
        out_specs=[
            pl.BlockSpec((_B, 1), lambda i: (0, 0)),
            pl.BlockSpec((_B, 1), lambda i: (0, 0)),
            pl.BlockSpec((_B, 1), lambda i: (0, 0)),
        ],
        out_shape=[
            jax.ShapeDtypeStruct((_B, 1), jnp.int32),
            jax.ShapeDtypeStruct((_B, 1), jnp.float32),
            jax.ShapeDtypeStruct((_B, 1), jnp.int32),
        ],
        scratch_shapes=[
            pltpu.VMEM((_B, _CH), jnp.int32),
            pltpu.VMEM((_B, _CH), jnp.float32),
            pltpu.VMEM((_B, _CH), jnp.float32),
            pltpu.VMEM((_B, _CH), jnp.int32),
            pltpu.VMEM((_B, _CH), jnp.float32),
            pltpu.VMEM((_B, _CH), jnp.int32),
            pltpu.VMEM((_B, _CH), jnp.float32),
        ],
        compiler_params=pltpu.CompilerParams(
            dimension_semantics=("arbitrary",)),
    )(actions.astype(jnp.int32), logits)
    return sample, log_probs, mode


# speedup vs baseline: 1.0115x; 1.0115x over previous
import numpy as np
import jax
import jax.numpy as jnp
from jax.experimental import pallas as pl
from jax.experimental.pallas import tpu as pltpu

_B = 32
_V = 1000000
_BLK = 8192
_NSTEPS = (_V + _BLK - 1) // _BLK
_CH = 256
_NCH = _BLK // _CH
_FULL = _V // _BLK
_TAILC = -(-(_V - _FULL * _BLK) // _CH)

_KS0 = 0
_KS1 = 42
_KS2 = (0x1BD11BDA ^ 42) & 0xFFFFFFFF
_ROTS = (13, 15, 26, 6, 17, 29, 16, 24, 13, 15, 26, 6, 17, 29, 16, 24,
         13, 15, 26, 6)
_INJ = {
    4: (_KS1, (_KS2 + 1) & 0xFFFFFFFF),
    8: (_KS2, (_KS0 + 2) & 0xFFFFFFFF),
    12: (_KS0, (_KS1 + 3) & 0xFFFFFFFF),
    16: (_KS1, (_KS2 + 4) & 0xFFFFFFFF),
    20: (_KS2, (_KS0 + 5) & 0xFFFFFFFF),
}

_TINY = np.float32(np.finfo(np.float32).tiny)


def _threefry_xor(c):
    x1 = c + jnp.uint32(_KS1)
    x0 = x1
    r = _ROTS[0]
    x1 = ((x1 << jnp.uint32(r)) | (x1 >> jnp.uint32(32 - r))) ^ x0
    for rnum, r in enumerate(_ROTS[1:], start=2):
        x0 = x0 + x1
        x1 = (x1 << jnp.uint32(r)) | (x1 >> jnp.uint32(32 - r))
        x1 = x1 ^ x0
        if rnum % 4 == 0:
            inj0, inj1 = _INJ[rnum]
            if inj0:
                x0 = x0 + jnp.uint32(inj0)
            x1 = x1 + jnp.uint32(inj1)
    return x0 ^ x1


def _neg_gumbel_from_bits(bits):
    fb = (bits >> jnp.uint32(9)) | jnp.uint32(0x3F800000)
    f = jax.lax.bitcast_convert_type(fb, jnp.float32) - jnp.float32(1.0)
    u = f + _TINY
    return jnp.log(-jnp.log(u))


def _body(a_ref, x_ref, samp_ref, lp_ref, mode_ref,
          rv_ref, s_ref, vm_ref, vi_ref, gm_ref, gi_ref, av_ref):
    step = pl.program_id(0)
    neg_inf = jnp.float32(-jnp.inf)

    @pl.when(step == 0)
    def _init():
        rv_ref[...] = (jax.lax.broadcasted_iota(jnp.int32, (_B, _CH), 0) * _V
                       + jax.lax.broadcasted_iota(jnp.int32, (_B, _CH), 1))
        s_ref[...] = jnp.zeros_like(s_ref)
        vm_ref[...] = jnp.full_like(vm_ref, neg_inf)
        vi_ref[...] = jnp.zeros_like(vi_ref)
        gm_ref[...] = jnp.full_like(gm_ref, neg_inf)
        gi_ref[...] = jnp.zeros_like(gi_ref)
        av_ref[...] = jnp.zeros_like(av_ref)

    base = step * _BLK
    rvl = rv_ref[...]
    ta = a_ref[...] + rv_ref[:, 0:1]

    def sweep(nch, masked):
        s = s_ref[...]
        vm = vm_ref[...]
        vi = vi_ref[...]
        gm = gm_ref[...]
        gi = gi_ref[...]
        av = av_ref[...]
        for c in range(nch):
            x = x_ref[:, c * _CH:(c + 1) * _CH]
            cnt = rvl + (base + c * _CH)
            if masked:
                x = jnp.where(
                    jax.lax.broadcasted_iota(jnp.int32, (_B, _CH), 1)
                    + (base + c * _CH) < _V, x, neg_inf)
            ng = _neg_gumbel_from_bits(_threefry_xor(cnt.astype(jnp.uint32)))
            y = x - ng
            gu = y > gm
            gm = jnp.where(gu, y, gm)
            gi = jnp.where(gu, cnt, gi)
            vu = x > vm
            vm = jnp.where(vu, x, vm)
            vi = jnp.where(vu, cnt, vi)
            s = s + jnp.exp(x)
            av = jnp.where(cnt == ta, x, av)
        s_ref[...] = s
        vm_ref[...] = vm
        vi_ref[...] = vi
        gm_ref[...] = gm
        gi_ref[...] = gi
        av_ref[...] = av

    @pl.when(step < _FULL)
    def _fast():
        sweep(_NCH, masked=False)

    @pl.when(step == _FULL)
    def _tail():
        sweep(_TAILC, masked=True)

    @pl.when(step == _NSTEPS - 1)
    def _fin():
        rv0 = rv_ref[:, 0:1]
        vm = vm_ref[...]
        bv = jnp.max(vm, axis=1, keepdims=True)
        mode_ref[...] = jnp.min(
            jnp.where(vm == bv, vi_ref[...], jnp.int32(0x7FFFFFFF)),
            axis=1, keepdims=True) - rv0
        gm = gm_ref[...]
        gv = jnp.max(gm, axis=1, keepdims=True)
        samp_ref[...] = jnp.min(
            jnp.where(gm == gv, gi_ref[...], jnp.int32(0x7FFFFFFF)),
            axis=1, keepdims=True) - rv0
        s = jnp.sum(s_ref[...], axis=1, keepdims=True)
        lp_ref[...] = (jnp.sum(av_ref[...], axis=1, keepdims=True)
                       - jnp.log(s))


def kernel(logits, actions):
    sample, log_probs, mode = pl.pallas_call(
        _body,
        grid=(_NSTEPS,),
        in_specs=[
            pl.BlockSpec((_B, 1), lambda i: (0, 0)),
            pl.BlockSpec((_B, _BLK), lambda i: (0, i)),
        ],
        out_specs=[
            pl.BlockSpec((_B, 1), lambda i: (0, 0)),
            pl.BlockSpec((_B, 1), lambda i: (0, 0)),
            pl.BlockSpec((_B, 1), lambda i: (0, 0)),
        ],
        out_shape=[
            jax.ShapeDtypeStruct((_B, 1), jnp.int32),
            jax.ShapeDtypeStruct((_B, 1), jnp.float32),
            jax.ShapeDtypeStruct((_B, 1), jnp.int32),
        ],
        scratch_shapes=[
            pltpu.VMEM((_B, _CH), jnp.int32),
            pltpu.VMEM((_B, _CH), jnp.float32),
            pltpu.VMEM((_B, _CH), jnp.float32),
            pltpu.VMEM((_B, _CH), jnp.int32),
            pltpu.VMEM((_B, _CH), jnp.float32),
            pltpu.VMEM((_B, _CH), jnp.int32),
            pltpu.VMEM((_B, _CH), jnp.float32),
        ],
        compiler_params=pltpu.CompilerParams(
            dimension_semantics=("arbitrary",)),
    )(actions.astype(jnp.int32), logits)
    return sample, log_probs, mode


# speedup vs baseline: 1.0163x; 1.0047x over previous
import numpy as np
import jax
import jax.numpy as jnp
from jax.experimental import pallas as pl
from jax.experimental.pallas import tpu as pltpu

_B = 32
_V = 1000000
_BLK = 16384
_NSTEPS = (_V + _BLK - 1) // _BLK
_CH = 256
_NCH = _BLK // _CH
_FULL = _V // _BLK
_TAILC = -(-(_V - _FULL * _BLK) // _CH)

_KS0 = 0
_KS1 = 42
_KS2 = (0x1BD11BDA ^ 42) & 0xFFFFFFFF
_ROTS = (13, 15, 26, 6, 17, 29, 16, 24, 13, 15, 26, 6, 17, 29, 16, 24,
         13, 15, 26, 6)
_INJ = {
    4: (_KS1, (_KS2 + 1) & 0xFFFFFFFF),
    8: (_KS2, (_KS0 + 2) & 0xFFFFFFFF),
    12: (_KS0, (_KS1 + 3) & 0xFFFFFFFF),
    16: (_KS1, (_KS2 + 4) & 0xFFFFFFFF),
    20: (_KS2, (_KS0 + 5) & 0xFFFFFFFF),
}

_TINY = np.float32(np.finfo(np.float32).tiny)


def _threefry_xor(c):
    x1 = c + jnp.uint32(_KS1)
    x0 = x1
    r = _ROTS[0]
    x1 = ((x1 << jnp.uint32(r)) | (x1 >> jnp.uint32(32 - r))) ^ x0
    for rnum, r in enumerate(_ROTS[1:], start=2):
        x0 = x0 + x1
        x1 = (x1 << jnp.uint32(r)) | (x1 >> jnp.uint32(32 - r))
        x1 = x1 ^ x0
        if rnum % 4 == 0:
            inj0, inj1 = _INJ[rnum]
            if inj0:
                x0 = x0 + jnp.uint32(inj0)
            x1 = x1 + jnp.uint32(inj1)
    return x0 ^ x1


def _neg_gumbel_from_bits(bits):
    fb = (bits >> jnp.uint32(9)) | jnp.uint32(0x3F800000)
    f = jax.lax.bitcast_convert_type(fb, jnp.float32) - jnp.float32(1.0)
    u = f + _TINY
    return jnp.log(-jnp.log(u))


def _body(a_ref, x_ref, samp_ref, lp_ref, mode_ref,
          rv_ref, s_ref, vm_ref, vi_ref, gm_ref, gi_ref, av_ref):
    step = pl.program_id(0)
    neg_inf = jnp.float32(-jnp.inf)

    @pl.when(step == 0)
    def _init():
        rv_ref[...] = (jax.lax.broadcasted_iota(jnp.int32, (_B, _CH), 0) * _V
                       + jax.lax.broadcasted_iota(jnp.int32, (_B, _CH), 1))
        s_ref[...] = jnp.zeros_like(s_ref)
        vm_ref[...] = jnp.full_like(vm_ref, neg_inf)
        vi_ref[...] = jnp.zeros_like(vi_ref)
        gm_ref[...] = jnp.full_like(gm_ref, neg_inf)
        gi_ref[...] = jnp.zeros_like(gi_ref)
        av_ref[...] = jnp.zeros_like(av_ref)

    base = step * _BLK
    rvl = rv_ref[...]
    ta = a_ref[...] + rv_ref[:, 0:1]

    def sweep(nch, masked):
        s = s_ref[...]
        vm = vm_ref[...]
        vi = vi_ref[...]
        gm = gm_ref[...]
        gi = gi_ref[...]
        av = av_ref[...]
        for c in range(nch):
            x = x_ref[:, c * _CH:(c + 1) * _CH]
            cnt = rvl + (base + c * _CH)
            if masked:
                x = jnp.where(
                    jax.lax.broadcasted_iota(jnp.int32, (_B, _CH), 1)
                    + (base + c * _CH) < _V, x, neg_inf)
            ng = _neg_gumbel_from_bits(_threefry_xor(cnt.astype(jnp.uint32)))
            y = x - ng
            gu = y > gm
            gm = jnp.where(gu, y, gm)
            gi = jnp.where(gu, cnt, gi)
            vu = x > vm
            vm = jnp.where(vu, x, vm)
            vi = jnp.where(vu, cnt, vi)
            s = s + jnp.exp(x)
            av = jnp.where(cnt == ta, x, av)
        s_ref[...] = s
        vm_ref[...] = vm
        vi_ref[...] = vi
        gm_ref[...] = gm
        gi_ref[...] = gi
        av_ref[...] = av

    @pl.when(step < _FULL)
    def _fast():
        sweep(_NCH, masked=False)

    @pl.when(step == _FULL)
    def _tail():
        sweep(_TAILC, masked=True)

    @pl.when(step == _NSTEPS - 1)
    def _fin():
        rv0 = rv_ref[:, 0:1]
        vm = vm_ref[...]
        bv = jnp.max(vm, axis=1, keepdims=True)
        mode_ref[...] = jnp.min(
            jnp.where(vm == bv, vi_ref[...], jnp.int32(0x7FFFFFFF)),
            axis=1, keepdims=True) - rv0
        gm = gm_ref[...]
        gv = jnp.max(gm, axis=1, keepdims=True)
        samp_ref[...] = jnp.min(
            jnp.where(gm == gv, gi_ref[...], jnp.int32(0x7FFFFFFF)),
            axis=1, keepdims=True) - rv0
        s = jnp.sum(s_ref[...], axis=1, keepdims=True)
        lp_ref[...] = (jnp.sum(av_ref[...], axis=1, keepdims=True)
                       - jnp.log(s))


def kernel(logits, actions):
    sample, log_probs, mode = pl.pallas_call(
        _body,
        grid=(_NSTEPS,),
        in_specs=[
            pl.BlockSpec((_B, 1), lambda i: (0, 0)),
            pl.BlockSpec((_B, _BLK), lambda i: (0, i)),
        ],
        out_specs=[
            pl.BlockSpec((_B, 1), lambda i: (0, 0)),
            pl.BlockSpec((_B, 1), lambda i: (0, 0)),
            pl.BlockSpec((_B, 1), lambda i: (0, 0)),
        ],
        out_shape=[
            jax.ShapeDtypeStruct((_B, 1), jnp.int32),
            jax.ShapeDtypeStruct((_B, 1), jnp.float32),
            jax.ShapeDtypeStruct((_B, 1), jnp.int32),
        ],
        scratch_shapes=[
            pltpu.VMEM((_B, _CH), jnp.int32),
            pltpu.VMEM((_B, _CH), jnp.float32),
            pltpu.VMEM((_B, _CH), jnp.float32),
            pltpu.VMEM((_B, _CH), jnp.int32),
            pltpu.VMEM((_B, _CH), jnp.float32),
            pltpu.VMEM((_B, _CH), jnp.int32),
            pltpu.VMEM((_B, _CH), jnp.float32),
        ],
        compiler_params=pltpu.CompilerParams(
            dimension_semantics=("arbitrary",)),
    )(actions.astype(jnp.int32), logits)
    return sample, log_probs, mode


# speedup vs baseline: 1.0181x; 1.0018x over previous
import numpy as np
import jax
import jax.numpy as jnp
from jax.experimental import pallas as pl
from jax.experimental.pallas import tpu as pltpu

_B = 32
_V = 1000000
_BLK = 16384
_NSTEPS = (_V + _BLK - 1) // _BLK
_CH = 384
_NCH = _BLK // _CH
_FULL = _V // _BLK
_TAILC = -(-(_V - _FULL * _BLK) // _CH)

_KS0 = 0
_KS1 = 42
_KS2 = (0x1BD11BDA ^ 42) & 0xFFFFFFFF
_ROTS = (13, 15, 26, 6, 17, 29, 16, 24, 13, 15, 26, 6, 17, 29, 16, 24,
         13, 15, 26, 6)
_INJ = {
    4: (_KS1, (_KS2 + 1) & 0xFFFFFFFF),
    8: (_KS2, (_KS0 + 2) & 0xFFFFFFFF),
    12: (_KS0, (_KS1 + 3) & 0xFFFFFFFF),
    16: (_KS1, (_KS2 + 4) & 0xFFFFFFFF),
    20: (_KS2, (_KS0 + 5) & 0xFFFFFFFF),
}

_TINY = np.float32(np.finfo(np.float32).tiny)


def _threefry_xor(c):
    x1 = c + jnp.uint32(_KS1)
    x0 = x1
    r = _ROTS[0]
    x1 = ((x1 << jnp.uint32(r)) | (x1 >> jnp.uint32(32 - r))) ^ x0
    for rnum, r in enumerate(_ROTS[1:], start=2):
        x0 = x0 + x1
        x1 = (x1 << jnp.uint32(r)) | (x1 >> jnp.uint32(32 - r))
        x1 = x1 ^ x0
        if rnum % 4 == 0:
            inj0, inj1 = _INJ[rnum]
            if inj0:
                x0 = x0 + jnp.uint32(inj0)
            x1 = x1 + jnp.uint32(inj1)
    return x0 ^ x1


def _neg_gumbel_from_bits(bits):
    fb = (bits >> jnp.uint32(9)) | jnp.uint32(0x3F800000)
    f = jax.lax.bitcast_convert_type(fb, jnp.float32) - jnp.float32(1.0)
    u = f + _TINY
    return jnp.log(-jnp.log(u))


def _body(a_ref, x_ref, samp_ref, lp_ref, mode_ref,
          rv_ref, s_ref, vm_ref, vi_ref, gm_ref, gi_ref, av_ref):
    step = pl.program_id(0)
    neg_inf = jnp.float32(-jnp.inf)

    @pl.when(step == 0)
    def _init():
        rv_ref[...] = (jax.lax.broadcasted_iota(jnp.int32, (_B, _CH), 0) * _V
                       + jax.lax.broadcasted_iota(jnp.int32, (_B, _CH), 1))
        s_ref[...] = jnp.zeros_like(s_ref)
        vm_ref[...] = jnp.full_like(vm_ref, neg_inf)
        vi_ref[...] = jnp.zeros_like(vi_ref)
        gm_ref[...] = jnp.full_like(gm_ref, neg_inf)
        gi_ref[...] = jnp.zeros_like(gi_ref)
        av_ref[...] = jnp.zeros_like(av_ref)

    base = step * _BLK
    rvl = rv_ref[...]
    ta = a_ref[...] + rv_ref[:, 0:1]

    def sweep(nch, masked):
        s = s_ref[...]
        vm = vm_ref[...]
        vi = vi_ref[...]
        gm = gm_ref[...]
        gi = gi_ref[...]
        av = av_ref[...]
        for c in range(nch):
            x = x_ref[:, c * _CH:(c + 1) * _CH]
            cnt = rvl + (base + c * _CH)
            if masked:
                x = jnp.where(
                    jax.lax.broadcasted_iota(jnp.int32, (_B, _CH), 1)
                    + (base + c * _CH) < _V, x, neg_inf)
            ng = _neg_gumbel_from_bits(_threefry_xor(cnt.astype(jnp.uint32)))
            y = x - ng
            gu = y > gm
            gm = jnp.where(gu, y, gm)
            gi = jnp.where(gu, cnt, gi)
            vu = x > vm
            vm = jnp.where(vu, x, vm)
            vi = jnp.where(vu, cnt, vi)
            s = s + jnp.exp(x)
            av = jnp.where(cnt == ta, x, av)
        s_ref[...] = s
        vm_ref[...] = vm
        vi_ref[...] = vi
        gm_ref[...] = gm
        gi_ref[...] = gi
        av_ref[...] = av

    @pl.when(step < _FULL)
    def _fast():
        sweep(_NCH, masked=False)

    @pl.when(step == _FULL)
    def _tail():
        sweep(_TAILC, masked=True)

    @pl.when(step == _NSTEPS - 1)
    def _fin():
        rv0 = rv_ref[:, 0:1]
        vm = vm_ref[...]
        bv = jnp.max(vm, axis=1, keepdims=True)
        mode_ref[...] = jnp.min(
            jnp.where(vm == bv, vi_ref[...], jnp.int32(0x7FFFFFFF)),
            axis=1, keepdims=True) - rv0
        gm = gm_ref[...]
        gv = jnp.max(gm, axis=1, keepdims=True)
        samp_ref[...] = jnp.min(
            jnp.where(gm == gv, gi_ref[...], jnp.int32(0x7FFFFFFF)),
            axis=1, keepdims=True) - rv0
        s = jnp.sum(s_ref[...], axis=1, keepdims=True)
        lp_ref[...] = (jnp.sum(av_ref[...], axis=1, keepdims=True)
                       - jnp.log(s))


def kernel(logits, actions):
    sample, log_probs, mode = pl.pallas_call(
        _body,
        grid=(_NSTEPS,),
        in_specs=[
            pl.BlockSpec((_B, 1), lambda i: (0, 0)),
            pl.BlockSpec((_B, _BLK), lambda i: (0, i)),
        ],
        out_specs=[
            pl.BlockSpec((_B, 1), lambda i: (0, 0)),
            pl.BlockSpec((_B, 1), lambda i: (0, 0)),
            pl.BlockSpec((_B, 1), lambda i: (0, 0)),
        ],
        out_shape=[
            jax.ShapeDtypeStruct((_B, 1), jnp.int32),
            jax.ShapeDtypeStruct((_B, 1), jnp.float32),
            jax.ShapeDtypeStruct((_B, 1), jnp.int32),
        ],
        scratch_shapes=[
            pltpu.VMEM((_B, _CH), jnp.int32),
            pltpu.VMEM((_B, _CH), jnp.float32),
            pltpu.VMEM((_B, _CH), jnp.float32),
            pltpu.VMEM((_B, _CH), jnp.int32),
            pltpu.VMEM((_B, _CH), jnp.float32),
            pltpu.VMEM((_B, _CH), jnp.int32),
            pltpu.VMEM((_B, _CH), jnp.float32),
        ],
        compiler_params=pltpu.CompilerParams(
            dimension_semantics=("arbitrary",)),
    )(actions.astype(jnp.int32), logits)
    return sample, log_probs, mode
